# Initial kernel scaffold; baseline (speedup 1.0000x reference)
import functools

import jax
import jax.numpy as jnp
from jax import lax
from jax.experimental import pallas as pl
from jax.experimental.pallas import tpu as pltpu
from jax.experimental.pallas import tpu_sc as plsc

NC = 2
NS = 16
NW = NC * NS

CHUNK = 2048


def _gather_body(x_hbm, table_hbm, out_hbm, idx_v, rows_v, sem, *, per_w, n_chunks):
  wid = lax.axis_index("s") * NC + lax.axis_index("c")
  base_w = wid * per_w

  def body(j, carry):
    base = base_w + j * CHUNK
    pltpu.sync_copy(x_hbm.at[pl.ds(base, CHUNK)], idx_v)
    pltpu.async_copy(table_hbm.at[idx_v], rows_v, sem).wait()
    pltpu.sync_copy(rows_v, out_hbm.at[pl.ds(base, CHUNK)])
    return carry

  lax.fori_loop(0, n_chunks, body, 0)


def kernel(x, table):
  batch, num_node = x.shape
  dim = table.shape[1]
  flat = x.reshape(-1).astype(jnp.int32)
  b = flat.shape[0]
  assert b % NW == 0
  per_w = b // NW
  assert per_w % CHUNK == 0
  n_chunks = per_w // CHUNK

  mesh = plsc.VectorSubcoreMesh(
      core_axis_name="c", subcore_axis_name="s", num_cores=NC, num_subcores=NS
  )
  out = pl.kernel(
      functools.partial(_gather_body, per_w=per_w, n_chunks=n_chunks),
      out_type=jax.ShapeDtypeStruct((b, dim), jnp.float32),
      mesh=mesh,
      scratch_types=[
          pltpu.VMEM((CHUNK,), jnp.int32),
          pltpu.VMEM((CHUNK, dim), jnp.float32),
          pltpu.SemaphoreType.DMA,
      ],
  )(flat, table)
  return out.reshape(batch, num_node, dim)


# speedup vs baseline: 2.4898x; 2.4898x over previous
import functools

import jax
import jax.numpy as jnp
from jax import lax
from jax.experimental import pallas as pl
from jax.experimental.pallas import tpu as pltpu
from jax.experimental.pallas import tpu_sc as plsc

NC = 2
NS = 16
NW = NC * NS

CHUNK = 2048


def _gather_body(x_hbm, table_hbm, out_hbm, idx_v, rows_v, sem, *, per_w, n_chunks):
  wid = lax.axis_index("s") * NC + lax.axis_index("c")
  base_w = wid * per_w

  def body(j, carry):
    base = base_w + j * CHUNK
    pltpu.sync_copy(x_hbm.at[pl.ds(base, CHUNK)], idx_v)
    pltpu.async_copy(table_hbm.at[idx_v], rows_v, sem).wait()
    pltpu.sync_copy(rows_v, out_hbm.at[pl.ds(base, CHUNK)])
    return carry

  lax.fori_loop(0, n_chunks, body, 0)


def kernel(x, table):
  batch, num_node = x.shape
  dim = table.shape[1]
  flat = x.reshape(-1).astype(jnp.int32)
  b = flat.shape[0]
  assert b % NW == 0
  per_w = b // NW
  assert per_w % CHUNK == 0
  n_chunks = per_w // CHUNK

  mesh = plsc.VectorSubcoreMesh(
      core_axis_name="c", subcore_axis_name="s", num_cores=NC, num_subcores=NS
  )
  out = pl.kernel(
      functools.partial(_gather_body, per_w=per_w, n_chunks=n_chunks),
      out_type=jax.ShapeDtypeStruct((b, dim), jnp.float32),
      mesh=mesh,
      scratch_types=[
          pltpu.VMEM((CHUNK,), jnp.int32),
          pltpu.VMEM((CHUNK, dim), jnp.float32),
          pltpu.SemaphoreType.DMA,
      ],
      compiler_params=pltpu.CompilerParams(use_tc_tiling_on_sc=False),
  )(flat, table)
  return out.reshape(batch, num_node, dim)


# speedup vs baseline: 2.5673x; 1.0311x over previous
import functools

import jax
import jax.numpy as jnp
from jax import lax
from jax.experimental import pallas as pl
from jax.experimental.pallas import tpu as pltpu
from jax.experimental.pallas import tpu_sc as plsc

NC = 2
NS = 16
NW = NC * NS

CHUNK = 1024
NBUF = 4


def _gather_body(x_hbm, table_hbm, out_hbm, idx_v, rows_v, *sems, per_w, n_groups):
  sem_i = sems[0:NBUF]
  sem_g = sems[NBUF:2 * NBUF]
  sem_s = sems[2 * NBUF:]
  wid = lax.axis_index("s") * NC + lax.axis_index("c")
  base_w = wid * per_w

  def start_idx(j, b):
    pltpu.async_copy(x_hbm.at[pl.ds(base_w + j * CHUNK, CHUNK)], idx_v.at[b],
                     sem_i[b])

  def start_gather(b):
    pltpu.async_copy(table_hbm.at[idx_v.at[b]], rows_v.at[b], sem_g[b])

  def start_store(j, b):
    pltpu.async_copy(rows_v.at[b], out_hbm.at[pl.ds(base_w + j * CHUNK, CHUNK)],
                     sem_s[b])

  def wait(c):
    c.wait()

  for b in range(NBUF):
    start_idx(b, b)
  for b in range(NBUF):
    pltpu.make_async_copy(x_hbm.at[pl.ds(0, CHUNK)], idx_v.at[b],
                          sem_i[b]).wait()
    start_gather(b)

  def body(g, carry):
    j0 = g * NBUF
    for b in range(NBUF):
      pltpu.make_async_copy(table_hbm.at[idx_v.at[b]], rows_v.at[b],
                            sem_g[b]).wait()
      start_store(j0 + b, b)
      start_idx(j0 + NBUF + b, b)
    for b in range(NBUF):
      pltpu.make_async_copy(x_hbm.at[pl.ds(0, CHUNK)], idx_v.at[b],
                            sem_i[b]).wait()
      pltpu.make_async_copy(rows_v.at[b], out_hbm.at[pl.ds(0, CHUNK)],
                            sem_s[b]).wait()
      start_gather(b)
    return carry

  lax.fori_loop(0, n_groups - 1, body, 0)

  j0 = (n_groups - 1) * NBUF
  for b in range(NBUF):
    pltpu.make_async_copy(table_hbm.at[idx_v.at[b]], rows_v.at[b],
                          sem_g[b]).wait()
    start_store(j0 + b, b)
  for b in range(NBUF):
    pltpu.make_async_copy(rows_v.at[b], out_hbm.at[pl.ds(0, CHUNK)],
                          sem_s[b]).wait()


def kernel(x, table):
  batch, num_node = x.shape
  dim = table.shape[1]
  flat = x.reshape(-1).astype(jnp.int32)
  b = flat.shape[0]
  assert b % NW == 0
  per_w = b // NW
  assert per_w % (CHUNK * NBUF) == 0
  n_groups = per_w // (CHUNK * NBUF)

  mesh = plsc.VectorSubcoreMesh(
      core_axis_name="c", subcore_axis_name="s", num_cores=NC, num_subcores=NS
  )
  out = pl.kernel(
      functools.partial(_gather_body, per_w=per_w, n_groups=n_groups),
      out_type=jax.ShapeDtypeStruct((b, dim), jnp.float32),
      mesh=mesh,
      scratch_types=(
          [pltpu.VMEM((NBUF, CHUNK), jnp.int32),
           pltpu.VMEM((NBUF, CHUNK, dim), jnp.float32)]
          + [pltpu.SemaphoreType.DMA] * (3 * NBUF)
      ),
      compiler_params=pltpu.CompilerParams(use_tc_tiling_on_sc=False),
  )(flat, table)
  return out.reshape(batch, num_node, dim)
